# Initial kernel scaffold; baseline (speedup 1.0000x reference)
import jax
import jax.numpy as jnp
from jax.experimental import pallas as pl


def kernel(cur_x, cache_x):
    raise NotImplementedError("write your pallas kernel here")


# speedup vs baseline: 1.4733x; 1.4733x over previous
import functools

import jax
import jax.numpy as jnp
from jax.experimental import pallas as pl
from jax.experimental.pallas import tpu as pltpu

CUR_H = 64
CUR_W = 64
CACHED_H = 32
CACHED_W = 32
TOPK = 2048


def _stats_kernel(cur_ref, sel_ref, mse_ref):
    L = cur_ref.shape[1]
    C = cur_ref.shape[2]
    TR = 128

    def mean_body(i, acc):
        blk = cur_ref[0, pl.ds(i * TR, TR), :]
        return acc + jnp.sum(blk, axis=0, keepdims=True)

    s = jax.lax.fori_loop(0, L // TR, mean_body,
                          jnp.zeros((1, C), jnp.float32))
    m = s * (1.0 / L)

    def mse_body(i, _):
        blk = cur_ref[0, pl.ds(i * TR, TR), :]
        d = blk - m
        mse_ref[pl.ds(i * TR, TR), :] = jnp.sum(d * d, axis=1, keepdims=True)
        return 0

    jax.lax.fori_loop(0, L // TR, mse_body, 0)

    msef = mse_ref[...].reshape(L // 128, 128)
    bits = jax.lax.bitcast_convert_type(msef, jnp.int32)

    def thresh_body(_, carry):
        lo, hi = carry
        mid = lo + (hi - lo) // 2
        cnt = jnp.sum((bits >= mid).astype(jnp.int32))
        take = cnt >= TOPK
        return (jnp.where(take, mid, lo), jnp.where(take, hi, mid))

    lo, hi = jax.lax.fori_loop(0, 31, thresh_body,
                               (jnp.int32(0), jnp.int32(0x7F800000)))
    thr = lo

    idx = (jax.lax.broadcasted_iota(jnp.int32, (L // 128, 128), 0) * 128
           + jax.lax.broadcasted_iota(jnp.int32, (L // 128, 128), 1))
    n_gt = jnp.sum((bits > thr).astype(jnp.int32))
    need = TOPK - n_gt
    eq = bits == thr

    def cut_body(_, carry):
        lo2, hi2 = carry
        mid = lo2 + (hi2 - lo2) // 2
        g = jnp.sum((eq & (idx < mid)).astype(jnp.int32))
        ok = g >= need
        return (jnp.where(ok, lo2, mid), jnp.where(ok, mid, hi2))

    _, cut = jax.lax.fori_loop(0, 13, cut_body,
                               (jnp.int32(0), jnp.int32(L)))
    cut = jnp.where(need > 0, cut, 0)

    sel = (bits > thr) | (eq & (idx < cut))
    sel_ref[0] = sel.astype(jnp.float32).reshape(L, 1)


def _select_kernel(sel_ref, cur_ref, cache_ref, out_ref):
    TB = cur_ref.shape[1]
    C = cur_ref.shape[2]
    rows = TB // 64
    cb = cache_ref[0]
    c4 = cb.reshape(rows // 2, 32, C)
    up = jnp.repeat(jnp.repeat(c4, 2, axis=0), 2, axis=1)
    up = up.reshape(TB, C)
    sel = sel_ref[0] > 0.5
    out_ref[0] = jnp.where(sel, cur_ref[0], up)


@jax.jit
def kernel(cur_x, cache_x):
    B, L, C = cur_x.shape
    TB = 512
    CB = TB // 4

    sel = pl.pallas_call(
        _stats_kernel,
        grid=(B,),
        in_specs=[pl.BlockSpec((1, L, C), lambda b: (b, 0, 0))],
        out_specs=pl.BlockSpec((1, L, 1), lambda b: (b, 0, 0)),
        out_shape=jax.ShapeDtypeStruct((B, L, 1), jnp.float32),
        scratch_shapes=[pltpu.VMEM((L, 1), jnp.float32)],
    )(cur_x)

    out = pl.pallas_call(
        _select_kernel,
        grid=(B, L // TB),
        in_specs=[
            pl.BlockSpec((1, TB, 1), lambda b, j: (b, j, 0)),
            pl.BlockSpec((1, TB, C), lambda b, j: (b, j, 0)),
            pl.BlockSpec((1, CB, C), lambda b, j: (b, j, 0)),
        ],
        out_specs=pl.BlockSpec((1, TB, C), lambda b, j: (b, j, 0)),
        out_shape=jax.ShapeDtypeStruct((B, L, C), jnp.float32),
    )(sel, cur_x, cache_x)
    return out


# speedup vs baseline: 1.5082x; 1.0236x over previous
import jax
import jax.numpy as jnp
from jax.experimental import pallas as pl
from jax.experimental.pallas import tpu as pltpu

TOPK = 2048


def _fused_kernel(cur_ref, cache_ref, out_ref, sel_ref, mse_ref):
    L = cur_ref.shape[1]
    C = cur_ref.shape[2]
    TR = 128
    TB = out_ref.shape[1]
    j = pl.program_id(1)

    @pl.when(j == 0)
    def _stats():
        def mean_body(i, acc):
            blk = cur_ref[0, pl.ds(i * TR, TR), :]
            return acc + jnp.sum(blk, axis=0, keepdims=True)

        s = jax.lax.fori_loop(0, L // TR, mean_body,
                              jnp.zeros((1, C), jnp.float32))
        m = s * (1.0 / L)

        def mse_body(i, _):
            blk = cur_ref[0, pl.ds(i * TR, TR), :]
            d = blk - m
            mse_ref[pl.ds(i * TR, TR), :] = jnp.sum(d * d, axis=1,
                                                    keepdims=True)
            return 0

        jax.lax.fori_loop(0, L // TR, mse_body, 0)

        msef = mse_ref[...].reshape(L // 128, 128)
        bits = jax.lax.bitcast_convert_type(msef, jnp.int32)

        def thresh_body(_, carry):
            lo, hi = carry
            mid = lo + (hi - lo) // 2
            cnt = jnp.sum((bits >= mid).astype(jnp.int32))
            take = cnt >= TOPK
            return (jnp.where(take, mid, lo), jnp.where(take, hi, mid))

        lo, hi = jax.lax.fori_loop(0, 31, thresh_body,
                                   (jnp.int32(0), jnp.int32(0x7F800000)))
        thr = lo

        idx = (jax.lax.broadcasted_iota(jnp.int32, (L // 128, 128), 0) * 128
               + jax.lax.broadcasted_iota(jnp.int32, (L // 128, 128), 1))
        n_gt = jnp.sum((bits > thr).astype(jnp.int32))
        need = TOPK - n_gt
        eq = bits == thr

        def cut_body(_, carry):
            lo2, hi2 = carry
            mid = lo2 + (hi2 - lo2) // 2
            g = jnp.sum((eq & (idx < mid)).astype(jnp.int32))
            ok = g >= need
            return (jnp.where(ok, lo2, mid), jnp.where(ok, mid, hi2))

        _, cut = jax.lax.fori_loop(0, 13, cut_body,
                                   (jnp.int32(0), jnp.int32(L)))
        cut = jnp.where(need > 0, cut, 0)

        sel = (bits > thr) | (eq & (idx < cut))
        sel_ref[...] = sel.astype(jnp.float32).reshape(L, 1)

    @pl.when(j > 0)
    def _select():
        blk = j - 1
        base = blk * TB
        rows = TB // 64
        cb = cache_ref[0]
        c4 = cb.reshape(rows // 2, 32, C)
        up = jnp.repeat(jnp.repeat(c4, 2, axis=0), 2, axis=1)
        up = up.reshape(TB, C)
        sel = sel_ref[pl.ds(base, TB), :] > 0.5
        out_ref[0] = jnp.where(sel, cur_ref[0, pl.ds(base, TB), :], up)


@jax.jit
def kernel(cur_x, cache_x):
    B, L, C = cur_x.shape
    Lc = cache_x.shape[1]
    TB = 512
    NB = L // TB

    out = pl.pallas_call(
        _fused_kernel,
        grid=(B, 1 + NB),
        in_specs=[
            pl.BlockSpec((1, L, C), lambda b, j: (b, 0, 0)),
            pl.BlockSpec((1, TB // 4, C),
                         lambda b, j: (b, jnp.maximum(j - 1, 0), 0)),
        ],
        out_specs=pl.BlockSpec(
            (1, TB, C),
            lambda b, j: (b, jnp.maximum(j - 1, 0), 0)),
        out_shape=jax.ShapeDtypeStruct((B, L, C), jnp.float32),
        scratch_shapes=[
            pltpu.VMEM((L, 1), jnp.float32),
            pltpu.VMEM((L, 1), jnp.float32),
        ],
    )(cur_x, cache_x)
    return out
